# baseline (speedup 1.0000x reference)
import functools

import jax
import jax.numpy as jnp
from jax import lax
from jax.experimental import pallas as pl
from jax.experimental.pallas import tpu as pltpu
from jax.experimental.pallas import tpu_sc as plsc

_B = 16384
_V = 1000000
_K = 64
_LANES = 16

_INFO = plsc.get_sparse_core_info()
_NC = _INFO.num_cores
_NS = _INFO.num_subcores
_NW = _NC * _NS
_BPW = _B // _NW
_NGRP = _BPW // _LANES
_RING = 8
_BLOCK_BYTES = _K * 128 * 4

_PARAMS = pltpu.CompilerParams(
    needs_layout_passes=False, disable_bounds_checks=True)


def _body(uidx_hbm, iidx_hbm, t_hbm, ut_hbm, it_hbm, tt_hbm, bias_hbm,
          out_hbm, ue_hbm, ie_hbm,
          idx_u, idx_i, u_ring, i_ring, t_v, tt_v, bias_v, out_v,
          s0, s1, s2, s3, s4, s5, s6, s7, sem, sem_i, sem2):
  slabs = (s0, s1, s2, s3, s4, s5, s6, s7)
  wid = lax.axis_index("s") * _NC + lax.axis_index("c")
  base = wid * _BPW

  pltpu.sync_copy(uidx_hbm.at[pl.ds(base, _BPW)], idx_u)
  pltpu.sync_copy(iidx_hbm.at[pl.ds(base, _BPW)], idx_i)

  rows4 = [k * _LANES + lax.iota(jnp.int32, _LANES) for k in range(4)]
  lane_iota = lax.iota(jnp.int32, _LANES)

  def fetch(table, r, slab, s):
    c0 = pl.multiple_of((r >> 7) * 128, 128)
    pltpu.async_copy(table.at[:, pl.ds(c0, 128)], slab, s)
    return r - c0

  def drain_fetch(slot, s):
    pltpu.make_async_copy(ut_hbm.at[:, pl.ds(0, 128)], slabs[slot],
                          s).wait()

  def extract(slot, rr):
    col = jnp.full((_LANES,), rr, jnp.int32)
    return [plsc.load_gather(slabs[slot], [rows4[k], col]) for k in range(4)]

  def drain_iouts(n):
    pltpu.make_async_copy(ie_hbm.at[pl.ds(0, n * _K)],
                          i_ring.at[pl.ds(0, n * _K)], sem2).wait()

  def do_i_row(slot, rr_sel, b_local, lane, dv):
    drain_fetch(slot, sem_i)
    vecs = extract(slot, rr_sel)
    im = (b_local & 31) * _K
    acc = u_ring[pl.ds(im, _LANES)] * vecs[0]
    for k in range(4):
      i_ring[pl.ds(im + k * _LANES, _LANES)] = vecs[k]
      if k:
        acc = acc + u_ring[pl.ds(im + k * _LANES, _LANES)] * vecs[k]
    d = jnp.sum(acc)
    pltpu.async_copy(i_ring.at[pl.ds(im, _K)],
                     ie_hbm.at[pl.ds((base + b_local) * _K, _K)], sem2)
    return jnp.where(lane_iota == lane, d, dv)

  def body(g, carry):
    rrs = list(carry[:_RING])
    dotvec = carry[_RING]

    lax.switch(jnp.clip(g, 0, 2),
               [lambda: None, lambda: drain_iouts(24), lambda: drain_iouts(32)])

    vu = idx_u[pl.ds(g * _LANES, _LANES)]
    vi = idx_i[pl.ds(g * _LANES, _LANES)]
    new_rrs = []
    for l in range(32):
      slot = l % _RING
      if l < _RING:
        lp, gb = l + 32 - _RING, g - 1
      else:
        lp, gb = l - _RING, g
      is_u = lp < 16
      b_local = gb * _LANES + (lp if is_u else lp - _LANES)
      rr_sel = rrs[l] if l < _RING else new_rrs[l - _RING]

      if is_u:
        def de_u(slot=slot, rr_sel=rr_sel, b_local=b_local):
          um = (b_local & 31) * _K
          drain_fetch(slot, sem)
          vecs = extract(slot, rr_sel)
          for k in range(4):
            u_ring[pl.ds(um + k * _LANES, _LANES)] = vecs[k]
          pltpu.async_copy(u_ring.at[pl.ds(um, _K)],
                           ue_hbm.at[pl.ds((base + b_local) * _K, _K)], sem2)
        de_u()
      else:
        lane = lp - _LANES
        def de_i(slot=slot, rr_sel=rr_sel, b_local=b_local, lane=lane,
                 dv=dotvec):
          return do_i_row(slot, rr_sel, b_local, lane, dv)
        if l < _RING:
          dotvec = lax.cond(g > 0, de_i, lambda dv=dotvec: dv)
        else:
          dotvec = de_i()

      if l == _RING - 1:
        def store_dots(dv=dotvec, g=g):
          out_v[pl.ds((g - 1) * _LANES, _LANES)] = dv
        lax.cond(g > 0, store_dots, lambda: None)

      if l < 16:
        new_rrs.append(fetch(ut_hbm, vu[l], slabs[slot], sem))
      else:
        new_rrs.append(fetch(it_hbm, vi[l - 16], slabs[slot], sem_i))
    return (*new_rrs[32 - _RING:], dotvec)

  carry0 = tuple(jnp.int32(0) for _ in range(_RING)) + (
      jnp.zeros((_LANES,), jnp.float32),)
  carry = lax.fori_loop(0, _NGRP, body, carry0)
  dotvec = carry[_RING]

  for l in range(_RING):
    b_local = _BPW - _RING + l
    dotvec = do_i_row(l % _RING, carry[l], b_local, 16 - _RING + l, dotvec)
  out_v[pl.ds((_NGRP - 1) * _LANES, _LANES)] = dotvec
  drain_iouts(32)
  drain_iouts(_RING)

  pltpu.sync_copy(tt_hbm, tt_v)
  pltpu.sync_copy(bias_hbm, bias_v)
  def t_step(i, acc):
    return acc + t_v[pl.ds(i * _LANES, _LANES)]
  n1v = jnp.zeros((_LANES,), jnp.int32)
  for c in range(4):
    pltpu.sync_copy(t_hbm.at[pl.ds(c * 4096, 4096)], t_v)
    n1v = lax.fori_loop(0, 4096 // _LANES, t_step, n1v)
  n1 = jnp.sum(n1v.astype(jnp.float32))
  sm0 = jnp.sum(tt_v[pl.ds(0, _LANES)] + tt_v[pl.ds(_LANES, _LANES)])
  sm1 = jnp.sum(tt_v[pl.ds(2 * _LANES, _LANES)] + tt_v[pl.ds(3 * _LANES, _LANES)])
  scalar = (jnp.float32(_B) - n1) * sm0 + n1 * sm1 + bias_v[:][0]
  def add_s(j, carry):
    out_v[pl.ds(j * _LANES, _LANES)] = (
        out_v[pl.ds(j * _LANES, _LANES)] + scalar)
    return carry
  lax.fori_loop(0, _NGRP, add_s, 0)

  pltpu.sync_copy(out_v, out_hbm.at[pl.ds(base, _BPW)])


@jax.jit
def _sc_forward(uidx, iidx, t, ut_t, it_t, tt, bias16):
  mesh = plsc.VectorSubcoreMesh(core_axis_name="c", subcore_axis_name="s")
  call = pl.kernel(
      _body,
      out_type=[
          jax.ShapeDtypeStruct((_B,), jnp.float32),
          jax.ShapeDtypeStruct((_B * _K,), jnp.float32),
          jax.ShapeDtypeStruct((_B * _K,), jnp.float32),
      ],
      mesh=mesh,
      compiler_params=_PARAMS,
      scratch_types=(
          [pltpu.VMEM((_BPW,), jnp.int32)] * 2
          + [pltpu.VMEM((32 * _K,), jnp.float32)]
          + [pltpu.VMEM((32 * _K,), jnp.float32)]
          + [pltpu.VMEM((4096,), jnp.int32)]
          + [pltpu.VMEM((4 * _LANES,), jnp.float32)]
          + [pltpu.VMEM((_LANES,), jnp.float32)]
          + [pltpu.VMEM((_BPW,), jnp.float32)]
          + [pltpu.VMEM((_K, 128), jnp.float32)] * _RING
          + [pltpu.SemaphoreType.DMA] * 3
      ),
  )
  return call(uidx, iidx, t, ut_t, it_t, tt, bias16)


def kernel(x, user_table, item_table, treatment_table, bias):
  x = x.astype(jnp.int32)
  uidx = x[:, 0]
  iidx = x[:, 1]
  t = x[:, 2]
  tt = treatment_table.reshape(4 * _LANES)
  bias16 = jnp.broadcast_to(bias.astype(jnp.float32), (_LANES,))
  out_flat, ue_flat, ie_flat = _sc_forward(
      uidx, iidx, t, user_table.T, item_table.T, tt, bias16)
  return (out_flat.reshape(_B, 1), ue_flat.reshape(_B, _K),
          ie_flat.reshape(_B, _K))


# speedup vs baseline: 1.0020x; 1.0020x over previous
import functools

import jax
import jax.numpy as jnp
from jax import lax
from jax.experimental import pallas as pl
from jax.experimental.pallas import tpu as pltpu
from jax.experimental.pallas import tpu_sc as plsc

_B = 16384
_V = 1000000
_K = 64
_LANES = 16

_INFO = plsc.get_sparse_core_info()
_NC = _INFO.num_cores
_NS = _INFO.num_subcores
_NW = _NC * _NS
_BPW = _B // _NW
_NGRP = _BPW // _LANES
_RING = 8
_BLOCK_BYTES = _K * 128 * 4

_PARAMS = pltpu.CompilerParams(
    needs_layout_passes=False, disable_bounds_checks=True)


def _body(uidx_hbm, iidx_hbm, t_hbm, ut_hbm, it_hbm, tt_hbm, bias_hbm,
          out_hbm, ue_hbm, ie_hbm,
          idx_u, idx_i, u_ring, i_ring, t_v, tt_v, bias_v, out_v,
          s0, s1, s2, s3, s4, s5, s6, s7, sem, sem_i, sem2):
  slabs = (s0, s1, s2, s3, s4, s5, s6, s7)
  wid = lax.axis_index("s") * _NC + lax.axis_index("c")
  base = wid * _BPW

  pltpu.sync_copy(uidx_hbm.at[pl.ds(base, _BPW)], idx_u)
  pltpu.sync_copy(iidx_hbm.at[pl.ds(base, _BPW)], idx_i)

  rows4 = [k * _LANES + lax.iota(jnp.int32, _LANES) for k in range(4)]
  lane_iota = lax.iota(jnp.int32, _LANES)

  def fetch(table, r, slab, s):
    c0 = pl.multiple_of((r >> 7) * 128, 128)
    pltpu.async_copy(table.at[:, pl.ds(c0, 128)], slab, s)
    return r - c0

  def drain_fetch(slot, s):
    pltpu.make_async_copy(ut_hbm.at[:, pl.ds(0, 128)], slabs[slot],
                          s).wait()

  def extract(slot, rr):
    col = jnp.full((_LANES,), rr, jnp.int32)
    return [plsc.load_gather(slabs[slot], [rows4[k], col]) for k in range(4)]

  def drain_iouts(n):
    pltpu.make_async_copy(ie_hbm.at[pl.ds(0, n * _K)],
                          i_ring.at[pl.ds(0, n * _K)], sem2).wait()

  def do_i_row(slot, rr_sel, b_local, lane, dv):
    drain_fetch(slot, sem_i)
    vecs = extract(slot, rr_sel)
    im = (b_local & 31) * _K
    acc = u_ring[pl.ds(im, _LANES)] * vecs[0]
    for k in range(4):
      i_ring[pl.ds(im + k * _LANES, _LANES)] = vecs[k]
      if k:
        acc = acc + u_ring[pl.ds(im + k * _LANES, _LANES)] * vecs[k]
    d = jnp.sum(acc)
    pltpu.async_copy(i_ring.at[pl.ds(im, _K)],
                     ie_hbm.at[pl.ds((base + b_local) * _K, _K)], sem2)
    return jnp.where(lane_iota == lane, d, dv)

  def body(g, carry):
    rrs = list(carry[:_RING])
    dotvec = carry[_RING]

    lax.switch(jnp.clip(g, 0, 2),
               [lambda: None, lambda: drain_iouts(24), lambda: drain_iouts(32)])

    vu = idx_u[pl.ds(g * _LANES, _LANES)]
    vi = idx_i[pl.ds(g * _LANES, _LANES)]
    new_rrs = []
    for l in range(32):
      slot = l % _RING
      if l < _RING:
        lp, gb = l + 32 - _RING, g - 1
      else:
        lp, gb = l - _RING, g
      is_u = lp < 16
      b_local = gb * _LANES + (lp if is_u else lp - _LANES)
      rr_sel = rrs[l] if l < _RING else new_rrs[l - _RING]

      if is_u:
        def de_u(slot=slot, rr_sel=rr_sel, b_local=b_local):
          um = (b_local & 31) * _K
          drain_fetch(slot, sem)
          vecs = extract(slot, rr_sel)
          for k in range(4):
            u_ring[pl.ds(um + k * _LANES, _LANES)] = vecs[k]
        de_u()
        if l == 23:
          pltpu.async_copy(
              u_ring.at[pl.ds((g & 1) * _LANES * _K, _LANES * _K)],
              ue_hbm.at[pl.ds((base + g * _LANES) * _K, _LANES * _K)], sem2)
      else:
        lane = lp - _LANES
        def de_i(slot=slot, rr_sel=rr_sel, b_local=b_local, lane=lane,
                 dv=dotvec):
          return do_i_row(slot, rr_sel, b_local, lane, dv)
        if l < _RING:
          dotvec = lax.cond(g > 0, de_i, lambda dv=dotvec: dv)
        else:
          dotvec = de_i()

      if l == _RING - 1:
        def store_dots(dv=dotvec, g=g):
          out_v[pl.ds((g - 1) * _LANES, _LANES)] = dv
        lax.cond(g > 0, store_dots, lambda: None)

      if l < 16:
        new_rrs.append(fetch(ut_hbm, vu[l], slabs[slot], sem))
      else:
        new_rrs.append(fetch(it_hbm, vi[l - 16], slabs[slot], sem_i))
    return (*new_rrs[32 - _RING:], dotvec)

  carry0 = tuple(jnp.int32(0) for _ in range(_RING)) + (
      jnp.zeros((_LANES,), jnp.float32),)
  carry = lax.fori_loop(0, _NGRP, body, carry0)
  dotvec = carry[_RING]

  for l in range(_RING):
    b_local = _BPW - _RING + l
    dotvec = do_i_row(l % _RING, carry[l], b_local, 16 - _RING + l, dotvec)
  out_v[pl.ds((_NGRP - 1) * _LANES, _LANES)] = dotvec
  drain_iouts(32)
  drain_iouts(_RING)

  pltpu.sync_copy(tt_hbm, tt_v)
  pltpu.sync_copy(bias_hbm, bias_v)
  def t_step(i, acc):
    return acc + t_v[pl.ds(i * _LANES, _LANES)]
  n1v = jnp.zeros((_LANES,), jnp.int32)
  for c in range(4):
    pltpu.sync_copy(t_hbm.at[pl.ds(c * 4096, 4096)], t_v)
    n1v = lax.fori_loop(0, 4096 // _LANES, t_step, n1v)
  n1 = jnp.sum(n1v.astype(jnp.float32))
  sm0 = jnp.sum(tt_v[pl.ds(0, _LANES)] + tt_v[pl.ds(_LANES, _LANES)])
  sm1 = jnp.sum(tt_v[pl.ds(2 * _LANES, _LANES)] + tt_v[pl.ds(3 * _LANES, _LANES)])
  scalar = (jnp.float32(_B) - n1) * sm0 + n1 * sm1 + bias_v[:][0]
  def add_s(j, carry):
    out_v[pl.ds(j * _LANES, _LANES)] = (
        out_v[pl.ds(j * _LANES, _LANES)] + scalar)
    return carry
  lax.fori_loop(0, _NGRP, add_s, 0)

  pltpu.sync_copy(out_v, out_hbm.at[pl.ds(base, _BPW)])


@jax.jit
def _sc_forward(uidx, iidx, t, ut_t, it_t, tt, bias16):
  mesh = plsc.VectorSubcoreMesh(core_axis_name="c", subcore_axis_name="s")
  call = pl.kernel(
      _body,
      out_type=[
          jax.ShapeDtypeStruct((_B,), jnp.float32),
          jax.ShapeDtypeStruct((_B * _K,), jnp.float32),
          jax.ShapeDtypeStruct((_B * _K,), jnp.float32),
      ],
      mesh=mesh,
      compiler_params=_PARAMS,
      scratch_types=(
          [pltpu.VMEM((_BPW,), jnp.int32)] * 2
          + [pltpu.VMEM((32 * _K,), jnp.float32)]
          + [pltpu.VMEM((32 * _K,), jnp.float32)]
          + [pltpu.VMEM((4096,), jnp.int32)]
          + [pltpu.VMEM((4 * _LANES,), jnp.float32)]
          + [pltpu.VMEM((_LANES,), jnp.float32)]
          + [pltpu.VMEM((_BPW,), jnp.float32)]
          + [pltpu.VMEM((_K, 128), jnp.float32)] * _RING
          + [pltpu.SemaphoreType.DMA] * 3
      ),
  )
  return call(uidx, iidx, t, ut_t, it_t, tt, bias16)


def kernel(x, user_table, item_table, treatment_table, bias):
  x = x.astype(jnp.int32)
  uidx = x[:, 0]
  iidx = x[:, 1]
  t = x[:, 2]
  tt = treatment_table.reshape(4 * _LANES)
  bias16 = jnp.broadcast_to(bias.astype(jnp.float32), (_LANES,))
  out_flat, ue_flat, ie_flat = _sc_forward(
      uidx, iidx, t, user_table.T, item_table.T, tt, bias16)
  return (out_flat.reshape(_B, 1), ue_flat.reshape(_B, _K),
          ie_flat.reshape(_B, _K))
